# baseline (speedup 1.0000x reference)
import jax
import jax.numpy as jnp
from jax.experimental import pallas as pl


def _identity_body(x_ref, o_ref):
    o_ref[...] = x_ref[...]


def _pdn_conv(h, row, col, edge_attr, W, b, w1, b1, w2, b2):
    n = h.shape[0]
    ew = jax.nn.sigmoid(jnp.maximum(edge_attr @ w1 + b1, 0.0) @ w2 + b2)[:, 0]
    sl = jnp.arange(n, dtype=row.dtype)
    r = jnp.concatenate([row, sl])
    c = jnp.concatenate([col, sl])
    w = jnp.concatenate([ew, jnp.ones((n,), ew.dtype)])
    deg = jax.ops.segment_sum(w, c, num_segments=n)
    dinv = jnp.where(deg > 0, 1.0 / jnp.sqrt(deg), 0.0)
    norm = dinv[r] * w * dinv[c]
    ht = h @ W
    out = jax.ops.segment_sum(norm[:, None] * ht[r], c, num_segments=n)
    return out + b


def _bn(h):
    mu = jnp.mean(h, axis=0)
    var = jnp.var(h, axis=0)
    return (h - mu) / jnp.sqrt(var + 1e-5)


def kernel(x, edge_index, batch, edge_attr, convW, convB, mlpW1, mlpB1, mlpW2, mlpB2, linW, linB):
    NG = 64
    row, col = edge_index[0], edge_index[1]

    def conv(i, h):
        return _pdn_conv(h, row, col, edge_attr, convW[i], convB[i], mlpW1[i], mlpB1[i], mlpW2[i], mlpB2[i])

    h = conv(0, x)
    x0 = h
    idx = 1
    for _ in range(2):
        h = conv(idx, jax.nn.relu(_bn(h)))
        idx += 1
    x1 = h + x0
    h = h + x0
    for _ in range(2):
        h = conv(idx, jax.nn.relu(_bn(h)))
        idx += 1
    x2 = h + x0 + x1
    h = h + x0 + x1
    for _ in range(2):
        h = conv(idx, jax.nn.relu(_bn(h)))
        idx += 1
    x3 = h + x0 + x1 + x2
    h = h + x0 + x1 + x2
    for _ in range(2):
        h = conv(idx, jax.nn.relu(_bn(h)))
        idx += 1
    x4 = jax.nn.relu(h + x0 + x1 + x2 + x3)
    pooled = jax.ops.segment_max(x4, batch, num_segments=NG)
    out = pooled @ linW + linB
    out = pl.pallas_call(
        _identity_body,
        out_shape=jax.ShapeDtypeStruct(out.shape, out.dtype),
    )(out)
    return out


# speedup vs baseline: 8.9262x; 8.9262x over previous
import functools

import jax
import jax.numpy as jnp
from jax import lax
from jax.experimental import pallas as pl
from jax.experimental.pallas import tpu as pltpu
from jax.experimental.pallas import tpu_sc as plsc

N = 10000
E = 320000
D = 128
ED = 16
NL = 9
NG = 64

NCORES = 2
NSUB = 16
NTILES = NCORES * NSUB
CK = 128
NCH = 79
EPT = NCH * CK
EPAD = NTILES * EPT
RPT = 624
_STRIPE_CHUNKS = [(0, 128), (128, 128), (256, 128), (384, 128), (512, 112)]

_HI = lax.Precision.HIGHEST


def _sigmoid(x):
    return jnp.where(x >= 0, 1.0 / (1.0 + jnp.exp(-x)), jnp.exp(x) / (1.0 + jnp.exp(x)))


def _ew_body(ea_ref, w1_ref, b1_ref, w2_ref, b2_ref, out_ref):
    ea = ea_ref[...]
    t = jnp.maximum(
        lax.dot_general(ea, w1_ref[...], (((1,), (0,)), ((), ())),
                        precision=_HI, preferred_element_type=jnp.float32)
        + b1_ref[...], 0.0)
    s = lax.dot_general(t, w2_ref[...], (((1,), (0,)), ((), ())),
                        precision=_HI, preferred_element_type=jnp.float32) + b2_ref[...]
    ew = _sigmoid(s)
    out_ref[...] = jnp.concatenate(
        [ew, jnp.zeros((ea.shape[0], 16 - NL), jnp.float32)], axis=1)


def _ew_call(edge_attr, W1cat, b1cat, W2blk, b2row):
    EB = 2000
    return pl.pallas_call(
        _ew_body,
        grid=(E // EB,),
        in_specs=[
            pl.BlockSpec((EB, ED), lambda i: (i, 0)),
            pl.BlockSpec((ED, NL * ED), lambda i: (0, 0)),
            pl.BlockSpec((1, NL * ED), lambda i: (0, 0)),
            pl.BlockSpec((NL * ED, NL), lambda i: (0, 0)),
            pl.BlockSpec((1, NL), lambda i: (0, 0)),
        ],
        out_specs=pl.BlockSpec((EB, 16), lambda i: (i, 0)),
        out_shape=jax.ShapeDtypeStruct((E, 16), jnp.float32),
    )(edge_attr, W1cat, b1cat, W2blk, b2row)


def _deg_body(col_h, ew_h, out_h, colv, ewv, eww, acc, sem):
    c = lax.axis_index("c")
    s = lax.axis_index("s")
    t = c * NSUB + s
    pltpu.sync_copy(col_h.at[t], colv)

    zero16 = jnp.zeros((16,), jnp.float32)

    def _z(i, _):
        for f in range(D // 16):
            eww[i, pl.ds(f * 16, 16)] = zero16
        return 0
    lax.fori_loop(0, CK, _z, 0)
    base = s * RPT
    for off, cnt in _STRIPE_CHUNKS:
        pltpu.sync_copy(eww.at[pl.ds(0, cnt)], acc.at[pl.ds(base + off, cnt)])

    @pl.when(s == NSUB - 1)
    def _ztail():
        pltpu.sync_copy(eww.at[pl.ds(0, 16)], acc.at[pl.ds(NSUB * RPT, 16)])
    plsc.subcore_barrier()

    def _chunk(j, _):
        pltpu.sync_copy(ew_h.at[pl.ds(t * EPT + j * CK, CK)], ewv)

        def _r(i, _):
            eww[i, pl.ds(0, 16)] = ewv[i, :]
            return 0
        lax.fori_loop(0, CK, _r, 0)
        pltpu.sync_copy(eww, acc.at[colv.at[j]], add=True)
        return 0
    lax.fori_loop(0, NCH, _chunk, 0)
    plsc.subcore_barrier()

    for off, cnt in _STRIPE_CHUNKS:
        pltpu.sync_copy(acc.at[pl.ds(base + off, cnt)], eww.at[pl.ds(0, cnt)])
        pltpu.sync_copy(eww.at[pl.ds(0, cnt)], out_h.at[c, pl.ds(base + off, cnt)])

    @pl.when(s == NSUB - 1)
    def _otail():
        pltpu.sync_copy(acc.at[pl.ds(NSUB * RPT, 16)], eww.at[pl.ds(0, 16)])
        pltpu.sync_copy(eww.at[pl.ds(0, 16)], out_h.at[c, pl.ds(NSUB * RPT, 16)])


def _deg_call(colp, ewp):
    mesh = plsc.VectorSubcoreMesh(core_axis_name="c", subcore_axis_name="s")
    f = functools.partial(
        pl.kernel, mesh=mesh,
        out_type=jax.ShapeDtypeStruct((NCORES, N, D), jnp.float32),
        scratch_types=[
            pltpu.VMEM((NCH, CK), jnp.int32),
            pltpu.VMEM((CK, 16), jnp.float32),
            pltpu.VMEM((CK, D), jnp.float32),
            pltpu.VMEM_SHARED((N, D), jnp.float32),
            pltpu.SemaphoreType.DMA,
        ],
    )(_deg_body)
    return f(colp, ewp)


def _dinv_body(degs_ref, x_ref, dinv_ref, gp_ref):
    deg = degs_ref[0][:, :16] + degs_ref[1][:, :16] + 1.0
    dinv = jnp.where(deg > 0, 1.0 / jnp.sqrt(deg), 0.0)
    dinv_ref[...] = dinv
    gp_ref[...] = dinv[:, 0:1] * x_ref[...]


def _dinv_call(degs, x):
    return pl.pallas_call(
        _dinv_body,
        out_shape=[jax.ShapeDtypeStruct((N, 16), jnp.float32),
                   jax.ShapeDtypeStruct((N, D), jnp.float32)],
    )(degs, x)


def _spmm_body(row_h, col_h, nrm_h, g_h, out_h, rowv, colv, nrmv, buf, acc, sem):
    c = lax.axis_index("c")
    s = lax.axis_index("s")
    t = c * NSUB + s
    pltpu.sync_copy(row_h.at[t], rowv)
    pltpu.sync_copy(col_h.at[t], colv)
    pltpu.sync_copy(nrm_h.at[t], nrmv)

    zero16 = jnp.zeros((16,), jnp.float32)

    def _z(i, _):
        for f in range(D // 16):
            buf[i, pl.ds(f * 16, 16)] = zero16
        return 0
    lax.fori_loop(0, CK, _z, 0)
    base = s * RPT
    for off, cnt in _STRIPE_CHUNKS:
        pltpu.sync_copy(buf.at[pl.ds(0, cnt)], acc.at[pl.ds(base + off, cnt)])

    @pl.when(s == NSUB - 1)
    def _ztail():
        pltpu.sync_copy(buf.at[pl.ds(0, 16)], acc.at[pl.ds(NSUB * RPT, 16)])
    plsc.subcore_barrier()

    def _chunk(j, _):
        pltpu.async_copy(g_h.at[rowv.at[j]], buf, sem).wait()

        def _eb(eb, _):
            nv16 = nrmv[j, pl.ds(eb * 16, 16)]
            for i in range(16):
                nv = jnp.full((16,), nv16[i], jnp.float32)
                e = eb * 16 + i
                for f in range(D // 16):
                    sl = pl.ds(f * 16, 16)
                    buf[e, sl] = buf[e, sl] * nv
            return 0
        lax.fori_loop(0, CK // 16, _eb, 0)
        pltpu.sync_copy(buf, acc.at[colv.at[j]], add=True)
        return 0
    lax.fori_loop(0, NCH, _chunk, 0)
    plsc.subcore_barrier()

    for off, cnt in _STRIPE_CHUNKS:
        pltpu.sync_copy(acc.at[pl.ds(base + off, cnt)], buf.at[pl.ds(0, cnt)])
        pltpu.sync_copy(buf.at[pl.ds(0, cnt)], out_h.at[c, pl.ds(base + off, cnt)])

    @pl.when(s == NSUB - 1)
    def _otail():
        pltpu.sync_copy(acc.at[pl.ds(NSUB * RPT, 16)], buf.at[pl.ds(0, 16)])
        pltpu.sync_copy(buf.at[pl.ds(0, 16)], out_h.at[c, pl.ds(NSUB * RPT, 16)])


def _spmm_call(rowp, colp, nrml, g):
    mesh = plsc.VectorSubcoreMesh(core_axis_name="c", subcore_axis_name="s")
    f = functools.partial(
        pl.kernel, mesh=mesh,
        out_type=jax.ShapeDtypeStruct((NCORES, N, D), jnp.float32),
        scratch_types=[
            pltpu.VMEM((NCH, CK), jnp.int32),
            pltpu.VMEM((NCH, CK), jnp.int32),
            pltpu.VMEM((NCH, CK), jnp.float32),
            pltpu.VMEM((CK, D), jnp.float32),
            pltpu.VMEM_SHARED((N, D), jnp.float32),
            pltpu.SemaphoreType.DMA,
        ],
    )(_spmm_body)
    return f(rowp, colp, nrml, g)


RB = 2000


def _conv_body(l, nskips, agg_ref, gp_ref, dinv_ref, w_ref, b_ref, *rest):
    skips = rest[:nskips]
    hs_ref = rest[nskips]
    dl = dinv_ref[...][:, l:l + 1]
    pre = dl * (agg_ref[0] + agg_ref[1] + gp_ref[...])
    h = lax.dot_general(pre, w_ref[...], (((1,), (0,)), ((), ())),
                        precision=_HI, preferred_element_type=jnp.float32) + b_ref[...]
    for sk in skips:
        h = h + sk[...]
    hs_ref[...] = h


def _conv_call(l, agg, gp, dinv, W, b, skips):
    nskips = len(skips)
    return pl.pallas_call(
        functools.partial(_conv_body, l, nskips),
        grid=(N // RB,),
        in_specs=[
            pl.BlockSpec((2, RB, D), lambda i: (0, i, 0)),
            pl.BlockSpec((RB, D), lambda i: (i, 0)),
            pl.BlockSpec((RB, 16), lambda i: (i, 0)),
            pl.BlockSpec((D, D), lambda i: (0, 0)),
            pl.BlockSpec((1, D), lambda i: (0, 0)),
        ] + [pl.BlockSpec((RB, D), lambda i: (i, 0))] * nskips,
        out_specs=pl.BlockSpec((RB, D), lambda i: (i, 0)),
        out_shape=jax.ShapeDtypeStruct((N, D), jnp.float32),
    )(agg, gp, dinv, W, b, *skips)


def _bn_body(l, hs_ref, dinv_ref, gpn_ref):
    h = hs_ref[...]
    mu = jnp.mean(h, axis=0, keepdims=True)
    var = jnp.mean((h - mu) ** 2, axis=0, keepdims=True)
    gn = jnp.maximum((h - mu) / jnp.sqrt(var + 1e-5), 0.0)
    gpn_ref[...] = dinv_ref[...][:, l + 1:l + 2] * gn


def _bn_call(l, hs, dinv):
    return pl.pallas_call(
        functools.partial(_bn_body, l),
        out_shape=jax.ShapeDtypeStruct((N, D), jnp.float32),
    )(hs, dinv)


def _pool_body(h_ref, batch_ref, lw_ref, lb_ref, out_ref, pooled_ref):
    x4 = jnp.maximum(h_ref[...], 0.0)
    bt = batch_ref[...]
    neg = jnp.float32(-jnp.inf)

    def _g(gi, _):
        m = bt == gi
        pooled_ref[pl.ds(gi, 1), :] = jnp.max(jnp.where(m, x4, neg), axis=0,
                                              keepdims=True)
        return 0
    lax.fori_loop(0, NG, _g, 0)
    pooled = pooled_ref[...]
    out_ref[...] = lax.dot_general(pooled, lw_ref[...], (((1,), (0,)), ((), ())),
                                   precision=_HI, preferred_element_type=jnp.float32) + lb_ref[...]


def _pool_call(h, batch2d, linW, linB2d):
    return pl.pallas_call(
        _pool_body,
        out_shape=jax.ShapeDtypeStruct((NG, 16), jnp.float32),
        scratch_shapes=[pltpu.VMEM((NG, D), jnp.float32)],
    )(h, batch2d, linW, linB2d)


def kernel(x, edge_index, batch, edge_attr, convW, convB, mlpW1, mlpB1, mlpW2,
           mlpB2, linW, linB):
    row, col = edge_index[0], edge_index[1]
    rowp = jnp.pad(row, (0, EPAD - E)).reshape(NTILES, NCH, CK)
    colp = jnp.pad(col, (0, EPAD - E)).reshape(NTILES, NCH, CK)

    W1cat = mlpW1.transpose(1, 0, 2).reshape(ED, NL * ED)
    b1cat = mlpB1.reshape(1, NL * ED)
    W2blk = (mlpW2[:, :, 0][:, :, None] * jnp.eye(NL, dtype=jnp.float32)[:, None, :]
             ).reshape(NL * ED, NL)
    b2row = mlpB2[:, 0].reshape(1, NL)

    ew = _ew_call(edge_attr, W1cat, b1cat, W2blk, b2row)
    ewp = jnp.pad(ew, ((0, EPAD - E), (0, 0)))
    ewt = ewp.T.reshape(16, NTILES, NCH, CK)
    degs = _deg_call(colp, ewp)
    dinv, gp = _dinv_call(degs, x)

    batch2d = batch.reshape(N, 1)
    linB2d = linB.reshape(1, 16)

    snaps = []
    skip_at = {0: 0, 2: 1, 4: 2, 6: 3}
    for l in range(8):
        agg = _spmm_call(rowp, colp, ewt[l], gp)
        skips = list(snaps) if l in skip_at else []
        hs = _conv_call(l, agg, gp, dinv, convW[l], convB[l].reshape(1, D), skips)
        gp = _bn_call(l, hs, dinv)
        if l in skip_at:
            snaps.append(hs)
    agg = _spmm_call(rowp, colp, ewt[8], gp)
    h8 = _conv_call(8, agg, gp, dinv, convW[8], convB[8].reshape(1, D), snaps)
    return _pool_call(h8, batch2d, linW, linB2d)


# speedup vs baseline: 10.7352x; 1.2027x over previous
import functools

import jax
import jax.numpy as jnp
from jax import lax
from jax.experimental import pallas as pl
from jax.experimental.pallas import tpu as pltpu
from jax.experimental.pallas import tpu_sc as plsc

N = 10000
E = 320000
D = 128
ED = 16
NL = 9
NG = 64

NCORES = 2
NSUB = 16
NTILES = NCORES * NSUB
CK = 128
NCH = 79
EPT = NCH * CK
EPAD = NTILES * EPT
RPT = 624
_STRIPE_CHUNKS = [(0, 128), (128, 128), (256, 128), (384, 128), (512, 112)]

_HI = lax.Precision.HIGHEST


def _sigmoid(x):
    return jnp.where(x >= 0, 1.0 / (1.0 + jnp.exp(-x)), jnp.exp(x) / (1.0 + jnp.exp(x)))


def _ew_body(ea_ref, w1_ref, b1_ref, w2_ref, b2_ref, out_ref):
    ea = ea_ref[...]
    t = jnp.maximum(
        lax.dot_general(ea, w1_ref[...], (((1,), (0,)), ((), ())),
                        precision=_HI, preferred_element_type=jnp.float32)
        + b1_ref[...], 0.0)
    s = lax.dot_general(t, w2_ref[...], (((1,), (0,)), ((), ())),
                        precision=_HI, preferred_element_type=jnp.float32) + b2_ref[...]
    ew = _sigmoid(s)
    out_ref[...] = jnp.concatenate(
        [ew, jnp.zeros((ea.shape[0], 16 - NL), jnp.float32)], axis=1)


def _ew_call(edge_attr, W1cat, b1cat, W2blk, b2row):
    EB = 2000
    return pl.pallas_call(
        _ew_body,
        grid=(E // EB,),
        in_specs=[
            pl.BlockSpec((EB, ED), lambda i: (i, 0)),
            pl.BlockSpec((ED, NL * ED), lambda i: (0, 0)),
            pl.BlockSpec((1, NL * ED), lambda i: (0, 0)),
            pl.BlockSpec((NL * ED, NL), lambda i: (0, 0)),
            pl.BlockSpec((1, NL), lambda i: (0, 0)),
        ],
        out_specs=pl.BlockSpec((EB, 16), lambda i: (i, 0)),
        out_shape=jax.ShapeDtypeStruct((E, 16), jnp.float32),
    )(edge_attr, W1cat, b1cat, W2blk, b2row)


def _deg_body(col_h, ew_h, out_h, colv, ewv, eww, acc, sem):
    c = lax.axis_index("c")
    s = lax.axis_index("s")
    t = c * NSUB + s
    pltpu.sync_copy(col_h.at[t], colv)

    zero16 = jnp.zeros((16,), jnp.float32)

    def _z(i, _):
        for f in range(D // 16):
            eww[i, pl.ds(f * 16, 16)] = zero16
        return 0
    lax.fori_loop(0, CK, _z, 0)
    base = s * RPT
    for off, cnt in _STRIPE_CHUNKS:
        pltpu.sync_copy(eww.at[pl.ds(0, cnt)], acc.at[pl.ds(base + off, cnt)])

    @pl.when(s == NSUB - 1)
    def _ztail():
        pltpu.sync_copy(eww.at[pl.ds(0, 16)], acc.at[pl.ds(NSUB * RPT, 16)])
    plsc.subcore_barrier()

    def _chunk(j, _):
        pltpu.sync_copy(ew_h.at[pl.ds(t * EPT + j * CK, CK)], ewv)

        @plsc.parallel_loop(0, CK, unroll=4)
        def _r(i):
            eww[i, pl.ds(0, 16)] = ewv[i, :]
        pltpu.sync_copy(eww, acc.at[colv.at[j]], add=True)
        return 0
    lax.fori_loop(0, NCH, _chunk, 0)
    plsc.subcore_barrier()

    for off, cnt in _STRIPE_CHUNKS:
        pltpu.sync_copy(acc.at[pl.ds(base + off, cnt)], eww.at[pl.ds(0, cnt)])
        pltpu.sync_copy(eww.at[pl.ds(0, cnt)], out_h.at[c, pl.ds(base + off, cnt)])

    @pl.when(s == NSUB - 1)
    def _otail():
        pltpu.sync_copy(acc.at[pl.ds(NSUB * RPT, 16)], eww.at[pl.ds(0, 16)])
        pltpu.sync_copy(eww.at[pl.ds(0, 16)], out_h.at[c, pl.ds(NSUB * RPT, 16)])


def _deg_call(colp, ewp):
    mesh = plsc.VectorSubcoreMesh(core_axis_name="c", subcore_axis_name="s")
    f = functools.partial(
        pl.kernel, mesh=mesh,
        out_type=jax.ShapeDtypeStruct((NCORES, N, D), jnp.float32),
        scratch_types=[
            pltpu.VMEM((NCH, CK), jnp.int32),
            pltpu.VMEM((CK, 16), jnp.float32),
            pltpu.VMEM((CK, D), jnp.float32),
            pltpu.VMEM_SHARED((N, D), jnp.float32),
            pltpu.SemaphoreType.DMA,
        ],
    )(_deg_body)
    return f(colp, ewp)


def _dinv_body(degs_ref, x_ref, dinv_ref, gp_ref):
    deg = degs_ref[0][:, :16] + degs_ref[1][:, :16] + 1.0
    dinv = jnp.where(deg > 0, 1.0 / jnp.sqrt(deg), 0.0)
    dinv_ref[...] = dinv
    gp_ref[...] = dinv[:, 0:1] * x_ref[...]


def _dinv_call(degs, x):
    return pl.pallas_call(
        _dinv_body,
        out_shape=[jax.ShapeDtypeStruct((N, 16), jnp.float32),
                   jax.ShapeDtypeStruct((N, D), jnp.float32)],
    )(degs, x)


def _spmm_body(pk_h, nrm_h, g_h, out_h, meta, nmeta, buf0, buf1, buf2, acc,
               msem, gsem, ssem):
    c = lax.axis_index("c")
    s = lax.axis_index("s")
    t = c * NSUB + s
    bufs = (buf0, buf1, buf2)

    zero16 = jnp.zeros((16,), jnp.float32)

    def _z(i, _):
        for f in range(D // 16):
            buf0[i, pl.ds(f * 16, 16)] = zero16
        return 0
    lax.fori_loop(0, CK, _z, 0)
    base = s * RPT
    for off, cnt in _STRIPE_CHUNKS:
        pltpu.sync_copy(buf0.at[pl.ds(0, cnt)], acc.at[pl.ds(base + off, cnt)])

    @pl.when(s == NSUB - 1)
    def _ztail():
        pltpu.sync_copy(buf0.at[pl.ds(0, 16)], acc.at[pl.ds(NSUB * RPT, 16)])
    plsc.subcore_barrier()

    def _im(j):
        pltpu.async_copy(pk_h.at[t, j], meta.at[lax.rem(j, 4)], msem)
        pltpu.async_copy(nrm_h.at[t, j], nmeta.at[lax.rem(j, 4)], msem)

    def _wm(j):
        pltpu.make_async_copy(pk_h.at[t, j], meta.at[lax.rem(j, 4)], msem).wait()
        pltpu.make_async_copy(nrm_h.at[t, j], nmeta.at[lax.rem(j, 4)], msem).wait()

    def _ig(j, b):
        pltpu.async_copy(g_h.at[meta.at[lax.rem(j, 4), 0]], b, gsem)

    def _wg(j, b):
        pltpu.make_async_copy(g_h.at[meta.at[lax.rem(j, 4), 0]], b, gsem).wait()

    def _is(j, b):
        pltpu.async_copy(b, acc.at[meta.at[lax.rem(j, 4), 1]], ssem, add=True)

    def _ws(j, b):
        pltpu.make_async_copy(b, acc.at[meta.at[lax.rem(j, 4), 1]], ssem).wait()

    def _scale(j, b):
        sl_j = lax.rem(j, 4)

        @plsc.parallel_loop(0, CK // 16, unroll=1)
        def _eb(eb):
            nv16 = nmeta[sl_j, 0, pl.ds(eb * 16, 16)]
            for i in range(16):
                nv = jnp.full((16,), nv16[i], jnp.float32)
                e = eb * 16 + i
                for f in range(D // 16):
                    sl = pl.ds(f * 16, 16)
                    b[e, sl] = b[e, sl] * nv

    _im(0)
    _im(1)
    _im(2)
    _wm(0)
    _ig(0, buf0)
    _im(3)
    _wm(1)
    _ig(1, buf1)
    _wg(0, buf0)
    _scale(0, buf0)
    _is(0, buf0)
    _wm(2)
    _ig(2, buf2)
    _wg(1, buf1)
    _scale(1, buf1)
    _is(1, buf1)

    def _steady(q, _):
        j0 = 3 * q + 2
        for r in range(3):
            j = j0 + r
            bj = bufs[(2 + r) % 3]
            bfree = bufs[r]
            _ws(j - 2, bfree)
            _im(j + 2)
            _wm(j + 1)
            _ig(j + 1, bfree)
            _wg(j, bj)
            _scale(j, bj)
            _is(j, bj)
        return 0
    lax.fori_loop(0, (NCH - 4) // 3, _steady, 0)
    _ws(NCH - 4, buf0)
    _wm(NCH - 1)
    _ig(NCH - 1, buf0)
    _wg(NCH - 2, buf2)
    _scale(NCH - 2, buf2)
    _is(NCH - 2, buf2)
    _ws(NCH - 3, buf1)
    _wg(NCH - 1, buf0)
    _scale(NCH - 1, buf0)
    _is(NCH - 1, buf0)
    _ws(NCH - 2, buf2)
    _ws(NCH - 1, buf0)

    plsc.subcore_barrier()

    for off, cnt in _STRIPE_CHUNKS:
        pltpu.sync_copy(acc.at[pl.ds(base + off, cnt)], buf0.at[pl.ds(0, cnt)])
        pltpu.sync_copy(buf0.at[pl.ds(0, cnt)], out_h.at[c, pl.ds(base + off, cnt)])

    @pl.when(s == NSUB - 1)
    def _otail():
        pltpu.sync_copy(acc.at[pl.ds(NSUB * RPT, 16)], buf1.at[pl.ds(0, 16)])
        pltpu.sync_copy(buf1.at[pl.ds(0, 16)], out_h.at[c, pl.ds(NSUB * RPT, 16)])


def _spmm_call(pk, nrm, g):
    mesh = plsc.VectorSubcoreMesh(core_axis_name="c", subcore_axis_name="s")
    f = functools.partial(
        pl.kernel, mesh=mesh,
        out_type=jax.ShapeDtypeStruct((NCORES, N, D), jnp.float32),
        scratch_types=[
            pltpu.VMEM((4, 2, CK), jnp.int32),
            pltpu.VMEM((4, 1, CK), jnp.float32),
            pltpu.VMEM((CK, D), jnp.float32),
            pltpu.VMEM((CK, D), jnp.float32),
            pltpu.VMEM((CK, D), jnp.float32),
            pltpu.VMEM_SHARED((N, D), jnp.float32),
            pltpu.SemaphoreType.DMA,
            pltpu.SemaphoreType.DMA,
            pltpu.SemaphoreType.DMA,
        ],
    )(_spmm_body)
    return f(pk, nrm, g)


RB = 2000


def _conv_body(l, nskips, agg_ref, gp_ref, dinv_ref, w_ref, b_ref, *rest):
    skips = rest[:nskips]
    hs_ref = rest[nskips]
    dl = dinv_ref[...][:, l:l + 1]
    pre = dl * (agg_ref[0] + agg_ref[1] + gp_ref[...])
    h = lax.dot_general(pre, w_ref[...], (((1,), (0,)), ((), ())),
                        precision=_HI, preferred_element_type=jnp.float32) + b_ref[...]
    for sk in skips:
        h = h + sk[...]
    hs_ref[...] = h


def _conv_call(l, agg, gp, dinv, W, b, skips):
    nskips = len(skips)
    return pl.pallas_call(
        functools.partial(_conv_body, l, nskips),
        grid=(N // RB,),
        in_specs=[
            pl.BlockSpec((2, RB, D), lambda i: (0, i, 0)),
            pl.BlockSpec((RB, D), lambda i: (i, 0)),
            pl.BlockSpec((RB, 16), lambda i: (i, 0)),
            pl.BlockSpec((D, D), lambda i: (0, 0)),
            pl.BlockSpec((1, D), lambda i: (0, 0)),
        ] + [pl.BlockSpec((RB, D), lambda i: (i, 0))] * nskips,
        out_specs=pl.BlockSpec((RB, D), lambda i: (i, 0)),
        out_shape=jax.ShapeDtypeStruct((N, D), jnp.float32),
    )(agg, gp, dinv, W, b, *skips)


def _bn_body(l, hs_ref, dinv_ref, gpn_ref):
    h = hs_ref[...]
    mu = jnp.mean(h, axis=0, keepdims=True)
    var = jnp.mean((h - mu) ** 2, axis=0, keepdims=True)
    gn = jnp.maximum((h - mu) / jnp.sqrt(var + 1e-5), 0.0)
    gpn_ref[...] = dinv_ref[...][:, l + 1:l + 2] * gn


def _bn_call(l, hs, dinv):
    return pl.pallas_call(
        functools.partial(_bn_body, l),
        out_shape=jax.ShapeDtypeStruct((N, D), jnp.float32),
    )(hs, dinv)


def _pool_body(h_ref, batch_ref, lw_ref, lb_ref, out_ref, pooled_ref):
    x4 = jnp.maximum(h_ref[...], 0.0)
    bt = batch_ref[...]
    neg = jnp.float32(-jnp.inf)

    def _g(gi, _):
        m = bt == gi
        pooled_ref[pl.ds(gi, 1), :] = jnp.max(jnp.where(m, x4, neg), axis=0,
                                              keepdims=True)
        return 0
    lax.fori_loop(0, NG, _g, 0)
    pooled = pooled_ref[...]
    out_ref[...] = lax.dot_general(pooled, lw_ref[...], (((1,), (0,)), ((), ())),
                                   precision=_HI, preferred_element_type=jnp.float32) + lb_ref[...]


def _pool_call(h, batch2d, linW, linB2d):
    return pl.pallas_call(
        _pool_body,
        out_shape=jax.ShapeDtypeStruct((NG, 16), jnp.float32),
        scratch_shapes=[pltpu.VMEM((NG, D), jnp.float32)],
    )(h, batch2d, linW, linB2d)


def kernel(x, edge_index, batch, edge_attr, convW, convB, mlpW1, mlpB1, mlpW2,
           mlpB2, linW, linB):
    row, col = edge_index[0], edge_index[1]
    rowp = jnp.pad(row, (0, EPAD - E)).reshape(NTILES, NCH, CK)
    colp = jnp.pad(col, (0, EPAD - E)).reshape(NTILES, NCH, CK)

    W1cat = mlpW1.transpose(1, 0, 2).reshape(ED, NL * ED)
    b1cat = mlpB1.reshape(1, NL * ED)
    W2blk = (mlpW2[:, :, 0][:, :, None] * jnp.eye(NL, dtype=jnp.float32)[:, None, :]
             ).reshape(NL * ED, NL)
    b2row = mlpB2[:, 0].reshape(1, NL)

    ew = _ew_call(edge_attr, W1cat, b1cat, W2blk, b2row)
    ewp = jnp.pad(ew, ((0, EPAD - E), (0, 0)))
    ewt = ewp.T.reshape(16, NTILES, NCH, CK)
    pk = jnp.stack([rowp, colp], axis=2)
    nrm9 = ewt[:NL].reshape(NL, NTILES, NCH, 1, CK)
    degs = _deg_call(colp, ewp)
    dinv, gp = _dinv_call(degs, x)

    batch2d = batch.reshape(N, 1)
    linB2d = linB.reshape(1, 16)

    snaps = []
    skip_at = {0: 0, 2: 1, 4: 2, 6: 3}
    for l in range(8):
        agg = _spmm_call(pk, nrm9[l], gp)
        skips = list(snaps) if l in skip_at else []
        hs = _conv_call(l, agg, gp, dinv, convW[l], convB[l].reshape(1, D), skips)
        gp = _bn_call(l, hs, dinv)
        if l in skip_at:
            snaps.append(hs)
    agg = _spmm_call(pk, nrm9[8], gp)
    h8 = _conv_call(8, agg, gp, dinv, convW[8], convB[8].reshape(1, D), snaps)
    return _pool_call(h8, batch2d, linW, linB2d)


# speedup vs baseline: 10.8615x; 1.0118x over previous
import functools

import jax
import jax.numpy as jnp
from jax import lax
from jax.experimental import pallas as pl
from jax.experimental.pallas import tpu as pltpu
from jax.experimental.pallas import tpu_sc as plsc

N = 10000
E = 320000
D = 128
ED = 16
NL = 9
NG = 64

NCORES = 2
NSUB = 16
NTILES = NCORES * NSUB
CK = 128
NCH = 79
EPT = NCH * CK
EPAD = NTILES * EPT
RPT = 624
_STRIPE_CHUNKS = [(0, 128), (128, 128), (256, 128), (384, 128), (512, 112)]

_HI = lax.Precision.HIGHEST


def _sigmoid(x):
    return jnp.where(x >= 0, 1.0 / (1.0 + jnp.exp(-x)), jnp.exp(x) / (1.0 + jnp.exp(x)))


def _ew_body(ea_ref, w1_ref, b1_ref, w2_ref, b2_ref, out_ref):
    ea = ea_ref[...]
    t = jnp.maximum(
        lax.dot_general(ea, w1_ref[...], (((1,), (0,)), ((), ())),
                        precision=_HI, preferred_element_type=jnp.float32)
        + b1_ref[...], 0.0)
    s = lax.dot_general(t, w2_ref[...], (((1,), (0,)), ((), ())),
                        precision=_HI, preferred_element_type=jnp.float32) + b2_ref[...]
    ew = _sigmoid(s)
    out_ref[...] = jnp.concatenate(
        [ew, jnp.zeros((ea.shape[0], 16 - NL), jnp.float32)], axis=1)


def _ew_call(edge_attr, W1cat, b1cat, W2blk, b2row):
    EB = 2000
    return pl.pallas_call(
        _ew_body,
        grid=(E // EB,),
        in_specs=[
            pl.BlockSpec((EB, ED), lambda i: (i, 0)),
            pl.BlockSpec((ED, NL * ED), lambda i: (0, 0)),
            pl.BlockSpec((1, NL * ED), lambda i: (0, 0)),
            pl.BlockSpec((NL * ED, NL), lambda i: (0, 0)),
            pl.BlockSpec((1, NL), lambda i: (0, 0)),
        ],
        out_specs=pl.BlockSpec((EB, 16), lambda i: (i, 0)),
        out_shape=jax.ShapeDtypeStruct((E, 16), jnp.float32),
    )(edge_attr, W1cat, b1cat, W2blk, b2row)


def _deg_body(col_h, ew_h, out_h, colv, ewv, ewv1, eww, eww1, acc, lsem, ssem):
    c = lax.axis_index("c")
    s = lax.axis_index("s")
    t = c * NSUB + s
    pltpu.sync_copy(col_h.at[t], colv)

    zero16 = jnp.zeros((16,), jnp.float32)

    def _z(i, _):
        for f in range(D // 16):
            eww[i, pl.ds(f * 16, 16)] = zero16
            eww1[i, pl.ds(f * 16, 16)] = zero16
        return 0
    lax.fori_loop(0, CK, _z, 0)
    base = s * RPT
    for off, cnt in _STRIPE_CHUNKS:
        pltpu.sync_copy(eww.at[pl.ds(0, cnt)], acc.at[pl.ds(base + off, cnt)])

    @pl.when(s == NSUB - 1)
    def _ztail():
        pltpu.sync_copy(eww.at[pl.ds(0, 16)], acc.at[pl.ds(NSUB * RPT, 16)])
    plsc.subcore_barrier()

    def _il(j, bv):
        pltpu.async_copy(ew_h.at[pl.ds(t * (EPT // 8) + j * (CK // 8), CK // 8)],
                         bv, lsem)

    def _wl(j, bv):
        pltpu.make_async_copy(
            ew_h.at[pl.ds(t * (EPT // 8) + j * (CK // 8), CK // 8)], bv, lsem).wait()

    def _is(j, bw):
        pltpu.async_copy(bw, acc.at[colv.at[j]], ssem, add=True)

    def _ws(j, bw):
        pltpu.make_async_copy(bw, acc.at[colv.at[j]], ssem).wait()

    def _stage(bv, bw):
        @plsc.parallel_loop(0, CK // 8, unroll=2)
        def _r(r):
            for k in range(8):
                bw[r * 8 + k, pl.ds(0, 16)] = bv[r, pl.ds(k * 16, 16)]

    _il(0, ewv)
    _il(1, ewv1)
    _wl(0, ewv)
    _stage(ewv, eww)
    _is(0, eww)
    _il(2, ewv)
    _wl(1, ewv1)
    _stage(ewv1, eww1)
    _is(1, eww1)

    def _pair(q, _):
        j0 = 2 * q + 2
        _ws(j0 - 2, eww)
        _il(j0 + 1, ewv1)
        _wl(j0, ewv)
        _stage(ewv, eww)
        _is(j0, eww)
        j1 = j0 + 1
        _ws(j1 - 2, eww1)
        _il(j1 + 1, ewv)
        _wl(j1, ewv1)
        _stage(ewv1, eww1)
        _is(j1, eww1)
        return 0
    lax.fori_loop(0, (NCH - 3) // 2, _pair, 0)
    _ws(NCH - 3, eww)
    _wl(NCH - 1, ewv)
    _stage(ewv, eww)
    _is(NCH - 1, eww)
    _ws(NCH - 2, eww1)
    _ws(NCH - 1, eww)
    plsc.subcore_barrier()

    for off, cnt in _STRIPE_CHUNKS:
        pltpu.sync_copy(acc.at[pl.ds(base + off, cnt)], eww.at[pl.ds(0, cnt)])
        pltpu.sync_copy(eww.at[pl.ds(0, cnt)], out_h.at[c, pl.ds(base + off, cnt)])

    @pl.when(s == NSUB - 1)
    def _otail():
        pltpu.sync_copy(acc.at[pl.ds(NSUB * RPT, 16)], eww1.at[pl.ds(0, 16)])
        pltpu.sync_copy(eww1.at[pl.ds(0, 16)], out_h.at[c, pl.ds(NSUB * RPT, 16)])


def _deg_call(colp, ewp):
    mesh = plsc.VectorSubcoreMesh(core_axis_name="c", subcore_axis_name="s")
    f = functools.partial(
        pl.kernel, mesh=mesh,
        out_type=jax.ShapeDtypeStruct((NCORES, N, D), jnp.float32),
        scratch_types=[
            pltpu.VMEM((NCH, CK), jnp.int32),
            pltpu.VMEM((CK // 8, CK), jnp.float32),
            pltpu.VMEM((CK // 8, CK), jnp.float32),
            pltpu.VMEM((CK, D), jnp.float32),
            pltpu.VMEM((CK, D), jnp.float32),
            pltpu.VMEM_SHARED((N, D), jnp.float32),
            pltpu.SemaphoreType.DMA,
            pltpu.SemaphoreType.DMA,
        ],
    )(_deg_body)
    return f(colp, ewp.reshape(EPAD // 8, CK))


def _dinv_body(degs_ref, x_ref, dinv_ref, gp_ref):
    deg = degs_ref[0][:, :16] + degs_ref[1][:, :16] + 1.0
    dinv = jnp.where(deg > 0, 1.0 / jnp.sqrt(deg), 0.0)
    dinv_ref[...] = dinv
    gp_ref[...] = dinv[:, 0:1] * x_ref[...]


def _dinv_call(degs, x):
    return pl.pallas_call(
        _dinv_body,
        out_shape=[jax.ShapeDtypeStruct((N, 16), jnp.float32),
                   jax.ShapeDtypeStruct((N, D), jnp.float32)],
    )(degs, x)


def _spmm_body(pk_h, nrm_h, g_h, out_h, meta, nmeta, buf0, buf1, buf2, acc,
               msem, gsem, ssem):
    c = lax.axis_index("c")
    s = lax.axis_index("s")
    t = c * NSUB + s
    bufs = (buf0, buf1, buf2)

    zero16 = jnp.zeros((16,), jnp.float32)

    def _z(i, _):
        for f in range(D // 16):
            buf0[i, pl.ds(f * 16, 16)] = zero16
        return 0
    lax.fori_loop(0, CK, _z, 0)
    base = s * RPT
    for off, cnt in _STRIPE_CHUNKS:
        pltpu.sync_copy(buf0.at[pl.ds(0, cnt)], acc.at[pl.ds(base + off, cnt)])

    @pl.when(s == NSUB - 1)
    def _ztail():
        pltpu.sync_copy(buf0.at[pl.ds(0, 16)], acc.at[pl.ds(NSUB * RPT, 16)])
    plsc.subcore_barrier()

    def _im(j):
        pltpu.async_copy(pk_h.at[t, j], meta.at[lax.rem(j, 4)], msem)
        pltpu.async_copy(nrm_h.at[t, j], nmeta.at[lax.rem(j, 4)], msem)

    def _wm(j):
        pltpu.make_async_copy(pk_h.at[t, j], meta.at[lax.rem(j, 4)], msem).wait()
        pltpu.make_async_copy(nrm_h.at[t, j], nmeta.at[lax.rem(j, 4)], msem).wait()

    def _ig(j, b):
        pltpu.async_copy(g_h.at[meta.at[lax.rem(j, 4), 0]], b, gsem)

    def _wg(j, b):
        pltpu.make_async_copy(g_h.at[meta.at[lax.rem(j, 4), 0]], b, gsem).wait()

    def _is(j, b):
        pltpu.async_copy(b, acc.at[meta.at[lax.rem(j, 4), 1]], ssem, add=True)

    def _ws(j, b):
        pltpu.make_async_copy(b, acc.at[meta.at[lax.rem(j, 4), 1]], ssem).wait()

    def _scale(j, b):
        sl_j = lax.rem(j, 4)

        @plsc.parallel_loop(0, CK // 16, unroll=2)
        def _eb(eb):
            nv16 = nmeta[sl_j, 0, pl.ds(eb * 16, 16)]
            for i in range(16):
                nv = jnp.full((16,), nv16[i], jnp.float32)
                e = eb * 16 + i
                for f in range(D // 16):
                    sl = pl.ds(f * 16, 16)
                    b[e, sl] = b[e, sl] * nv

    _im(0)
    _im(1)
    _im(2)
    _wm(0)
    _ig(0, buf0)
    _im(3)
    _wm(1)
    _ig(1, buf1)
    _wg(0, buf0)
    _scale(0, buf0)
    _is(0, buf0)
    _wm(2)
    _ig(2, buf2)
    _wg(1, buf1)
    _scale(1, buf1)
    _is(1, buf1)

    def _steady(q, _):
        j0 = 3 * q + 2
        for r in range(3):
            j = j0 + r
            bj = bufs[(2 + r) % 3]
            bfree = bufs[r]
            _ws(j - 2, bfree)
            _im(j + 2)
            _wm(j + 1)
            _ig(j + 1, bfree)
            _wg(j, bj)
            _scale(j, bj)
            _is(j, bj)
        return 0
    lax.fori_loop(0, (NCH - 4) // 3, _steady, 0)
    _ws(NCH - 4, buf0)
    _wm(NCH - 1)
    _ig(NCH - 1, buf0)
    _wg(NCH - 2, buf2)
    _scale(NCH - 2, buf2)
    _is(NCH - 2, buf2)
    _ws(NCH - 3, buf1)
    _wg(NCH - 1, buf0)
    _scale(NCH - 1, buf0)
    _is(NCH - 1, buf0)
    _ws(NCH - 2, buf2)
    _ws(NCH - 1, buf0)

    plsc.subcore_barrier()

    for off, cnt in _STRIPE_CHUNKS:
        pltpu.sync_copy(acc.at[pl.ds(base + off, cnt)], buf0.at[pl.ds(0, cnt)])
        pltpu.sync_copy(buf0.at[pl.ds(0, cnt)], out_h.at[c, pl.ds(base + off, cnt)])

    @pl.when(s == NSUB - 1)
    def _otail():
        pltpu.sync_copy(acc.at[pl.ds(NSUB * RPT, 16)], buf1.at[pl.ds(0, 16)])
        pltpu.sync_copy(buf1.at[pl.ds(0, 16)], out_h.at[c, pl.ds(NSUB * RPT, 16)])


def _spmm_call(pk, nrm, g):
    mesh = plsc.VectorSubcoreMesh(core_axis_name="c", subcore_axis_name="s")
    f = functools.partial(
        pl.kernel, mesh=mesh,
        out_type=jax.ShapeDtypeStruct((NCORES, N, D), jnp.float32),
        scratch_types=[
            pltpu.VMEM((4, 2, CK), jnp.int32),
            pltpu.VMEM((4, 1, CK), jnp.float32),
            pltpu.VMEM((CK, D), jnp.float32),
            pltpu.VMEM((CK, D), jnp.float32),
            pltpu.VMEM((CK, D), jnp.float32),
            pltpu.VMEM_SHARED((N, D), jnp.float32),
            pltpu.SemaphoreType.DMA,
            pltpu.SemaphoreType.DMA,
            pltpu.SemaphoreType.DMA,
        ],
    )(_spmm_body)
    return f(pk, nrm, g)


RB = 2000


def _conv_body(l, nskips, agg_ref, gp_ref, dinv_ref, w_ref, b_ref, *rest):
    skips = rest[:nskips]
    hs_ref = rest[nskips]
    dl = dinv_ref[...][:, l:l + 1]
    pre = dl * (agg_ref[0] + agg_ref[1] + gp_ref[...])
    h = lax.dot_general(pre, w_ref[...], (((1,), (0,)), ((), ())),
                        precision=_HI, preferred_element_type=jnp.float32) + b_ref[...]
    for sk in skips:
        h = h + sk[...]
    hs_ref[...] = h


def _conv_call(l, agg, gp, dinv, W, b, skips):
    nskips = len(skips)
    return pl.pallas_call(
        functools.partial(_conv_body, l, nskips),
        grid=(N // RB,),
        in_specs=[
            pl.BlockSpec((2, RB, D), lambda i: (0, i, 0)),
            pl.BlockSpec((RB, D), lambda i: (i, 0)),
            pl.BlockSpec((RB, 16), lambda i: (i, 0)),
            pl.BlockSpec((D, D), lambda i: (0, 0)),
            pl.BlockSpec((1, D), lambda i: (0, 0)),
        ] + [pl.BlockSpec((RB, D), lambda i: (i, 0))] * nskips,
        out_specs=pl.BlockSpec((RB, D), lambda i: (i, 0)),
        out_shape=jax.ShapeDtypeStruct((N, D), jnp.float32),
    )(agg, gp, dinv, W, b, *skips)


def _bn_body(l, hs_ref, dinv_ref, gpn_ref):
    h = hs_ref[...]
    mu = jnp.mean(h, axis=0, keepdims=True)
    var = jnp.mean((h - mu) ** 2, axis=0, keepdims=True)
    gn = jnp.maximum((h - mu) / jnp.sqrt(var + 1e-5), 0.0)
    gpn_ref[...] = dinv_ref[...][:, l + 1:l + 2] * gn


def _bn_call(l, hs, dinv):
    return pl.pallas_call(
        functools.partial(_bn_body, l),
        out_shape=jax.ShapeDtypeStruct((N, D), jnp.float32),
    )(hs, dinv)


def _pool_body(h_ref, batch_ref, lw_ref, lb_ref, out_ref, pooled_ref):
    x4 = jnp.maximum(h_ref[...], 0.0)
    bt = batch_ref[...]
    neg = jnp.float32(-jnp.inf)

    def _g(gi, _):
        m = bt == gi
        pooled_ref[pl.ds(gi, 1), :] = jnp.max(jnp.where(m, x4, neg), axis=0,
                                              keepdims=True)
        return 0
    lax.fori_loop(0, NG, _g, 0)
    pooled = pooled_ref[...]
    out_ref[...] = lax.dot_general(pooled, lw_ref[...], (((1,), (0,)), ((), ())),
                                   precision=_HI, preferred_element_type=jnp.float32) + lb_ref[...]


def _pool_call(h, batch2d, linW, linB2d):
    return pl.pallas_call(
        _pool_body,
        out_shape=jax.ShapeDtypeStruct((NG, 16), jnp.float32),
        scratch_shapes=[pltpu.VMEM((NG, D), jnp.float32)],
    )(h, batch2d, linW, linB2d)


def kernel(x, edge_index, batch, edge_attr, convW, convB, mlpW1, mlpB1, mlpW2,
           mlpB2, linW, linB):
    row, col = edge_index[0], edge_index[1]
    rowp = jnp.pad(row, (0, EPAD - E)).reshape(NTILES, NCH, CK)
    colp = jnp.pad(col, (0, EPAD - E)).reshape(NTILES, NCH, CK)

    W1cat = mlpW1.transpose(1, 0, 2).reshape(ED, NL * ED)
    b1cat = mlpB1.reshape(1, NL * ED)
    W2blk = (mlpW2[:, :, 0][:, :, None] * jnp.eye(NL, dtype=jnp.float32)[:, None, :]
             ).reshape(NL * ED, NL)
    b2row = mlpB2[:, 0].reshape(1, NL)

    ew = _ew_call(edge_attr, W1cat, b1cat, W2blk, b2row)
    ewp = jnp.pad(ew, ((0, EPAD - E), (0, 0)))
    ewt = ewp.T.reshape(16, NTILES, NCH, CK)
    pk = jnp.stack([rowp, colp], axis=2)
    nrm9 = ewt[:NL].reshape(NL, NTILES, NCH, 1, CK)
    degs = _deg_call(colp, ewp)
    dinv, gp = _dinv_call(degs, x)

    batch2d = batch.reshape(N, 1)
    linB2d = linB.reshape(1, 16)

    snaps = []
    skip_at = {0: 0, 2: 1, 4: 2, 6: 3}
    for l in range(8):
        agg = _spmm_call(pk, nrm9[l], gp)
        skips = list(snaps) if l in skip_at else []
        hs = _conv_call(l, agg, gp, dinv, convW[l], convB[l].reshape(1, D), skips)
        gp = _bn_call(l, hs, dinv)
        if l in skip_at:
            snaps.append(hs)
    agg = _spmm_call(pk, nrm9[8], gp)
    h8 = _conv_call(8, agg, gp, dinv, convW[8], convB[8].reshape(1, D), snaps)
    return _pool_call(h8, batch2d, linW, linB2d)
